# baseline (speedup 1.0000x reference)
import functools

import jax
import jax.numpy as jnp
from jax.experimental import pallas as pl
from jax.experimental.pallas import tpu as pltpu

EPS = 1e-5
BLOCK_S = 256


def _ln_kernel(x_ref, pos_ref, gamma_ref, beta_ref, out_ref):
    h = x_ref[...] + pos_ref[...][None]
    d_inv = 1.0 / h.shape[-1]
    s1 = jnp.sum(h, axis=-1, keepdims=True)
    s2 = jnp.sum(h * h, axis=-1, keepdims=True)
    mean = s1 * d_inv
    var = s2 * d_inv - mean * mean
    inv = jax.lax.rsqrt(var + EPS)
    out_ref[...] = (h - mean) * inv * gamma_ref[...] + beta_ref[...]


@functools.partial(jax.jit, static_argnames=())
def kernel(x, pos_table, ln_gamma, ln_beta):
    B, S, D = x.shape
    bs = min(BLOCK_S, S)
    grid = (S // bs,)
    return pl.pallas_call(
        _ln_kernel,
        grid=grid,
        in_specs=[
            pl.BlockSpec((B, bs, D), lambda s: (0, s, 0)),
            pl.BlockSpec((bs, D), lambda s: (s, 0)),
            pl.BlockSpec((D,), lambda s: (0,)),
            pl.BlockSpec((D,), lambda s: (0,)),
        ],
        out_specs=pl.BlockSpec((B, bs, D), lambda s: (0, s, 0)),
        out_shape=jax.ShapeDtypeStruct((B, S, D), x.dtype),
        compiler_params=pltpu.CompilerParams(
            dimension_semantics=("arbitrary",),
        ),
    )(x, pos_table[:S], ln_gamma, ln_beta)


# speedup vs baseline: 1.0409x; 1.0409x over previous
import functools

import jax
import jax.numpy as jnp
from jax.experimental import pallas as pl
from jax.experimental.pallas import tpu as pltpu

EPS = 1e-5
BLOCK_S = 512


def _ln_kernel(x_ref, pos_ref, gamma_ref, beta_ref, out_ref):
    h = x_ref[...] + pos_ref[...][None]
    d_inv = 1.0 / h.shape[-1]
    ones = jnp.ones((h.shape[-1], 8), dtype=jnp.float32)
    dn = (((2,), (0,)), ((), ()))
    s1 = jax.lax.dot_general(h, ones, dn,
                             preferred_element_type=jnp.float32)[..., :1]
    s2 = jax.lax.dot_general(h * h, ones, dn,
                             preferred_element_type=jnp.float32)[..., :1]
    mean = s1 * d_inv
    var = s2 * d_inv - mean * mean
    inv = jax.lax.rsqrt(var + EPS)
    out_ref[...] = (h - mean) * inv * gamma_ref[...] + beta_ref[...]


@functools.partial(jax.jit, static_argnames=())
def kernel(x, pos_table, ln_gamma, ln_beta):
    B, S, D = x.shape
    bs = min(BLOCK_S, S)
    grid = (S // bs,)
    return pl.pallas_call(
        _ln_kernel,
        grid=grid,
        in_specs=[
            pl.BlockSpec((B, bs, D), lambda s: (0, s, 0)),
            pl.BlockSpec((bs, D), lambda s: (s, 0)),
            pl.BlockSpec((D,), lambda s: (0,)),
            pl.BlockSpec((D,), lambda s: (0,)),
        ],
        out_specs=pl.BlockSpec((B, bs, D), lambda s: (0, s, 0)),
        out_shape=jax.ShapeDtypeStruct((B, S, D), x.dtype),
        compiler_params=pltpu.CompilerParams(
            dimension_semantics=("arbitrary",),
        ),
    )(x, pos_table[:S], ln_gamma, ln_beta)
